# baseline (speedup 1.0000x reference)
import jax
import jax.numpy as jnp
from jax.experimental import pallas as pl


def _vq_block(z_ref, e_ref, en_ref, kf_ref, zq_ref, idx_ref):
    TB, D = z_ref.shape[1], z_ref.shape[2]
    zb = z_ref[...].reshape(TB, D)
    e = e_ref[...]
    en = en_ref[...]
    kf = kf_ref[...]
    zn = jnp.sum(zb * zb, axis=-1, keepdims=True)
    dots = jnp.dot(zb, e.T, preferred_element_type=jnp.float32)
    d = zn + en[None, :] - 2.0 * dots
    m = jnp.min(d, axis=-1, keepdims=True)
    kiof = kf[None, :]
    idxf = jnp.min(jnp.where(d == m, kiof, float(d.shape[1])), axis=-1)
    idx_ref[...] = idxf.astype(jnp.int32).reshape(1, 1, TB)
    oh = (kiof == idxf[:, None]).astype(jnp.float32)
    zq = jnp.dot(oh, e, preferred_element_type=jnp.float32)
    zq_ref[...] = (zb + (zq - zb)).reshape(1, TB, D)


def kernel(z, emb_weight):
    B, T, D = z.shape
    K = emb_weight.shape[0]
    en = jnp.sum(emb_weight ** 2, axis=-1)
    kf = jnp.arange(K, dtype=jnp.float32)

    zq, idx = pl.pallas_call(
        _vq_block,
        grid=(B,),
        in_specs=[
            pl.BlockSpec((1, T, D), lambda i: (i, 0, 0)),
            pl.BlockSpec((K, D), lambda i: (0, 0)),
            pl.BlockSpec((K,), lambda i: (0,)),
            pl.BlockSpec((K,), lambda i: (0,)),
        ],
        out_specs=[
            pl.BlockSpec((1, T, D), lambda i: (i, 0, 0)),
            pl.BlockSpec((1, 1, T), lambda i: (i, 0, 0)),
        ],
        out_shape=[
            jax.ShapeDtypeStruct((B, T, D), jnp.float32),
            jax.ShapeDtypeStruct((B, 1, T), jnp.int32),
        ],
    )(z, emb_weight, en, kf)
    return zq, idx.reshape(B, T)


# speedup vs baseline: 1.1885x; 1.1885x over previous
import jax
import jax.numpy as jnp
from jax.experimental import pallas as pl

_TB = 4096


def _vq_block(z_ref, e_ref, en_ref, kf_ref, zq_ref, idx_ref):
    zb = z_ref[...]
    e = e_ref[...]
    en = en_ref[...]
    kf = kf_ref[...]
    zn = jnp.sum(zb * zb, axis=-1, keepdims=True)
    dots = jnp.dot(zb, e.T, preferred_element_type=jnp.float32)
    d = zn + en[None, :] - 2.0 * dots
    m = jnp.min(d, axis=-1, keepdims=True)
    kiof = kf[None, :]
    idxf = jnp.min(jnp.where(d == m, kiof, float(d.shape[1])), axis=-1)
    idx_ref[...] = idxf.astype(jnp.int32)
    oh = (kiof == idxf[:, None]).astype(jnp.float32)
    zq = jnp.dot(oh, e, preferred_element_type=jnp.float32)
    zq_ref[...] = zb + (zq - zb)


def kernel(z, emb_weight):
    B, T, D = z.shape
    K = emb_weight.shape[0]
    zf = z.reshape(B * T, D)
    en = jnp.sum(emb_weight ** 2, axis=-1)
    kf = jnp.arange(K, dtype=jnp.float32)
    grid = (B * T) // _TB

    zq, idx = pl.pallas_call(
        _vq_block,
        grid=(grid,),
        in_specs=[
            pl.BlockSpec((_TB, D), lambda i: (i, 0)),
            pl.BlockSpec((K, D), lambda i: (0, 0)),
            pl.BlockSpec((K,), lambda i: (0,)),
            pl.BlockSpec((K,), lambda i: (0,)),
        ],
        out_specs=[
            pl.BlockSpec((_TB, D), lambda i: (i, 0)),
            pl.BlockSpec((_TB,), lambda i: (i,)),
        ],
        out_shape=[
            jax.ShapeDtypeStruct((B * T, D), jnp.float32),
            jax.ShapeDtypeStruct((B * T,), jnp.int32),
        ],
    )(zf, emb_weight, en, kf)
    return zq.reshape(B, T, D), idx.reshape(B, T)
